# baseline (speedup 1.0000x reference)
import jax
import jax.numpy as jnp
import numpy as np
from jax.experimental import pallas as pl
from jax.experimental.pallas import tpu as pltpu

_G = 2
_R = 2
_CDIM = 4
_DPG = 512
_HALF_L = 4.0 * (1.0 + 1e-3) / 2.0
_HALF_W = 2.0
_BASIS = (1.0, 5.0, 25.0, 125.0)
_TT = 2048
_CT = 512
_LANES = 128


def _fsq_kernel(w_ref, b_ref, x_ref, o_ref):
    f32 = jnp.float32
    for kk in range(_TT // _CT):
        cs = slice(kk * _CT, (kk + 1) * _CT)
        for g in range(_G):
            accs = [None] * _CDIM
            for j in range(_DPG // 8):
                xs = x_ref[0, g * _DPG + 8 * j:g * _DPG + 8 * (j + 1), cs]
                xs = xs.astype(jnp.bfloat16).astype(f32)
                for c in range(_CDIM):
                    wv = w_ref[g * _CDIM + c, 8 * j:8 * (j + 1), :]
                    wt = jnp.tile(wv, (1, _CT // _LANES))
                    p = wt * xs
                    accs[c] = p if accs[c] is None else accs[c] + p
            for c in range(_CDIM):
                k = g * _CDIM + c
                z = jnp.sum(accs[c], axis=0, keepdims=True) + b_ref[k:k + 1, 0:1]
                r0 = jnp.round(jnp.tanh(z) * _HALF_L)
                resid = z - r0 * (1.0 / _HALF_W)
                r1 = jnp.round(jnp.tanh(resid * 4.0) * _HALF_L)
                i0 = (r0 + _HALF_W) * _BASIS[c]
                i1 = (r1 + _HALF_W) * _BASIS[c]
                if c == 0:
                    idx0, idx1 = i0, i1
                else:
                    idx0, idx1 = idx0 + i0, idx1 + i1
            o_ref[0, 2 * g, cs] = idx0[0].astype(jnp.int32)
            o_ref[0, 2 * g + 1, cs] = idx1[0].astype(jnp.int32)


def kernel(x, Win, b_in, Wout, b_out):
    del Wout, b_out
    B, D, T = x.shape
    wk = jnp.concatenate([Win[0], Win[1]], axis=0)
    wk = wk.astype(jnp.bfloat16).astype(jnp.float32)
    wrep = jnp.broadcast_to(wk[:, :, None], (_G * _CDIM, _DPG, _LANES))
    b8 = jnp.concatenate([b_in[0], b_in[1]]).reshape(_G * _CDIM, 1)
    grid = (B, T // _TT)
    out = pl.pallas_call(
        _fsq_kernel,
        grid=grid,
        in_specs=[
            pl.BlockSpec((_G * _CDIM, _DPG, _LANES), lambda bi, ti: (0, 0, 0)),
            pl.BlockSpec((_G * _CDIM, 1), lambda bi, ti: (0, 0)),
            pl.BlockSpec((1, D, _TT), lambda bi, ti: (bi, 0, ti)),
        ],
        out_specs=pl.BlockSpec((1, _G * _R, _TT), lambda bi, ti: (bi, 0, ti)),
        out_shape=jax.ShapeDtypeStruct((B, _G * _R, T), jnp.int32),
        compiler_params=pltpu.CompilerParams(
            dimension_semantics=("parallel", "parallel"),
        ),
    )(wrep, b8, x)
    return out
